# baseline (speedup 1.0000x reference)
import jax
import jax.numpy as jnp
import numpy as np
from jax import lax
from jax.experimental import pallas as pl
from jax.experimental.pallas import tpu as pltpu
from jax.experimental.pallas import tpu_sc as plsc

B, S, D, L, T = 4, 4096, 1024, 4096, 4
PD = 256
N = B * S
NC, NS = 2, 16
NW = NC * NS
PER_W = N // NW
P = 16
NCHUNK = PER_W // P
NBUF = 3


def _sc_body(x_hbm, pe0, pe1, pe2, pe3, idx_hbm, out_hbm,
             idx_v, rows_v, xbuf, semx, semo, semi):
    pes = [pe0, pe1, pe2, pe3]
    c = lax.axis_index("c")
    s = lax.axis_index("s")
    wid = s * NC + c
    base = wid * PER_W

    idx_cp = pltpu.make_async_copy(idx_hbm.at[:, pl.ds(base, PER_W)],
                                   idx_v, semi)
    idx_cp.start()

    def load_copies(cn, bn):
        cbase = base + cn * P
        cps = [pltpu.make_async_copy(x_hbm.at[pl.ds(cbase, P)],
                                     xbuf.at[bn], semx.at[bn])]
        for t in range(T):
            cps.append(pltpu.make_async_copy(
                pes[t].at[idx_v.at[t, pl.ds(cn * P, P)]],
                rows_v.at[bn, t], semx.at[bn]))
        return cps

    def store_copy(cn, bn):
        cbase = base + cn * P
        return pltpu.make_async_copy(xbuf.at[bn], out_hbm.at[pl.ds(cbase, P)],
                                     semo.at[bn])

    def start_loads(cn, bn):
        for cp in load_copies(cn, bn):
            cp.start()

    def wait_loads(cn, bn):
        for cp in load_copies(cn, bn):
            cp.wait()

    for k in range(NBUF):
        load_copies(k, k)[0].start()
    idx_cp.wait()
    for k in range(NBUF):
        for cp in load_copies(k, k)[1:]:
            cp.start()

    def chunk_body(ci, carry):
        b = lax.rem(ci, NBUF)
        cn = ci + NBUF - 1
        bn = lax.rem(cn, NBUF)

        @pl.when(jnp.logical_and(ci >= 1, cn < NCHUNK))
        def _():
            store_copy(cn - NBUF, bn).wait()
            start_loads(cn, bn)

        wait_loads(ci, b)

        xb = xbuf.at[b]
        rb = rows_v.at[b]

        himask = jnp.int32(-65536)
        @plsc.parallel_loop(0, P, 1, unroll=4)
        def add_body(p):
            for t in range(T):
                for g in range(PD // 32):
                    w = rb[t, p, pl.ds(g * 16, 16)]
                    lo = lax.bitcast_convert_type(
                        lax.shift_left(w, 16), jnp.float32)
                    hi = lax.bitcast_convert_type(w & himask, jnp.float32)
                    col = t * PD + g * 32
                    plsc.addupdate(xb.at[p, pl.ds(col, 16)], lo)
                    plsc.addupdate(xb.at[p, pl.ds(col + 16, 16)], hi)
        store_copy(ci, b).start()
        return carry

    lax.fori_loop(0, NCHUNK, chunk_body, 0, unroll=False)

    for k in range(NCHUNK - NBUF, NCHUNK):
        store_copy(k, k % NBUF).wait()


def kernel(x, pe0, pe1, pe2, pe3, indexes):
    x2 = x.reshape(N, D)
    idx = indexes.reshape(N, T).T

    perm = np.arange(PD).reshape(PD // 32, 2, 16).transpose(0, 2, 1).ravel()

    def pack(pe):
        pe_bf = pe[:, perm].astype(jnp.bfloat16)
        return lax.bitcast_convert_type(
            pe_bf.reshape(L, PD // 2, 2), jnp.int32)

    pe_p = [pack(pe) for pe in (pe0, pe1, pe2, pe3)]

    mesh = plsc.VectorSubcoreMesh(core_axis_name="c", subcore_axis_name="s")
    run = pl.kernel(
        _sc_body,
        out_type=jax.ShapeDtypeStruct((N, D), jnp.float32),
        mesh=mesh,
        scratch_types=[
            pltpu.VMEM((T, PER_W), jnp.int32),
            pltpu.VMEM((NBUF, T, P, PD // 2), jnp.int32),
            pltpu.VMEM((NBUF, P, D), jnp.float32),
            pltpu.SemaphoreType.DMA((NBUF,)),
            pltpu.SemaphoreType.DMA((NBUF,)),
            pltpu.SemaphoreType.DMA,
        ],
    )
    out = run(x2, pe_p[0], pe_p[1], pe_p[2], pe_p[3], idx)
    return out.reshape(B, S, D)


# speedup vs baseline: 1.9321x; 1.9321x over previous
import jax
import jax.numpy as jnp
from jax import lax
from jax.experimental import pallas as pl
from jax.experimental.pallas import tpu as pltpu
from jax.experimental.pallas import tpu_sc as plsc

B, S, D, L, T = 4, 4096, 1024, 4096, 4
PD = 256
N = B * S
NC, NS = 2, 16
NW = NC * NS
PER_W = N // NW
P = 16
NCHUNK = PER_W // P
NBUF = 3


def _sc_body(x_hbm, pe0, pe1, pe2, pe3, idx_hbm, out_hbm,
             idx_v, rows_v, xbuf, semx, semo, semi):
    pes = [pe0, pe1, pe2, pe3]
    c = lax.axis_index("c")
    s = lax.axis_index("s")
    wid = s * NC + c
    base = wid * PER_W

    idx_cp = pltpu.make_async_copy(idx_hbm.at[:, pl.ds(base, PER_W)],
                                   idx_v, semi)
    idx_cp.start()

    def load_copies(cn, bn):
        cbase = base + cn * P
        cps = [pltpu.make_async_copy(x_hbm.at[pl.ds(cbase, P)],
                                     xbuf.at[bn], semx.at[bn])]
        for t in range(T):
            cps.append(pltpu.make_async_copy(
                pes[t].at[idx_v.at[t, pl.ds(cn * P, P)]],
                rows_v.at[bn, t], semx.at[bn]))
        return cps

    def store_copy(cn, bn):
        cbase = base + cn * P
        return pltpu.make_async_copy(xbuf.at[bn], out_hbm.at[pl.ds(cbase, P)],
                                     semo.at[bn])

    def start_loads(cn, bn):
        for cp in load_copies(cn, bn):
            cp.start()

    def wait_loads(cn, bn):
        for cp in load_copies(cn, bn):
            cp.wait()

    for k in range(NBUF):
        load_copies(k, k)[0].start()
    idx_cp.wait()
    for k in range(NBUF):
        for cp in load_copies(k, k)[1:]:
            cp.start()

    def chunk_body(ci, carry):
        b = lax.rem(ci, NBUF)
        cn = ci + NBUF - 1
        bn = lax.rem(cn, NBUF)

        @pl.when(jnp.logical_and(ci >= 1, cn < NCHUNK))
        def _():
            store_copy(cn - NBUF, bn).wait()
            start_loads(cn, bn)

        wait_loads(ci, b)

        xb = xbuf.at[b]
        rb = rows_v.at[b]

        @plsc.parallel_loop(0, P, 1, unroll=4)
        def add_body(p):
            for t in range(T):
                for j in range(PD // 16):
                    col = t * PD + j * 16
                    rv = rb[t, p, pl.ds(j * 16, 16)]
                    plsc.addupdate(xb.at[p, pl.ds(col, 16)], rv)
        store_copy(ci, b).start()
        return carry

    lax.fori_loop(0, NCHUNK, chunk_body, 0, unroll=False)

    for k in range(NCHUNK - NBUF, NCHUNK):
        store_copy(k, k % NBUF).wait()


def kernel(x, pe0, pe1, pe2, pe3, indexes):
    x2 = x.reshape(N, D)
    idx = indexes.reshape(N, T).T

    mesh = plsc.VectorSubcoreMesh(core_axis_name="c", subcore_axis_name="s")
    run = pl.kernel(
        _sc_body,
        out_type=jax.ShapeDtypeStruct((N, D), jnp.float32),
        mesh=mesh,
        scratch_types=[
            pltpu.VMEM((T, PER_W), jnp.int32),
            pltpu.VMEM((NBUF, T, P, PD), jnp.float32),
            pltpu.VMEM((NBUF, P, D), jnp.float32),
            pltpu.SemaphoreType.DMA((NBUF,)),
            pltpu.SemaphoreType.DMA((NBUF,)),
            pltpu.SemaphoreType.DMA,
        ],
    )
    out = run(x2, pe0, pe1, pe2, pe3, idx)
    return out.reshape(B, S, D)
